# Initial kernel scaffold; baseline (speedup 1.0000x reference)
import functools

import jax
import jax.numpy as jnp
from jax import lax
from jax.experimental import pallas as pl
from jax.experimental.pallas import tpu as pltpu
from jax.experimental.pallas import tpu_sc as plsc

N = 10000
E = 320000
D = 128
G = 64

NC = 2
NS = 16
NW = NC * NS

CH = 80
EW = E // NW
NCH = EW // CH
ROWS_T = N // NS

RW = 320
RCH = 80


def _sc_mesh():
    return plsc.VectorSubcoreMesh(core_axis_name="c", subcore_axis_name="s")


@functools.partial(
    pl.kernel,
    out_type=jax.ShapeDtypeStruct((NC, N, 16), jnp.float32),
    mesh=_sc_mesh(),
    scratch_types=[
        pltpu.VMEM((CH,), jnp.int32),
        pltpu.VMEM((CH, 16), jnp.float32),
    ],
)
def _sc_degree(dst_hbm, zeros16_hbm, ones16_hbm, out_hbm, didx, ones_v):
    c = lax.axis_index("c")
    s = lax.axis_index("s")

    def inner(acc16):
        r0 = s * ROWS_T
        pltpu.sync_copy(zeros16_hbm.at[pl.ds(r0, ROWS_T)], acc16.at[pl.ds(r0, ROWS_T)])
        pltpu.sync_copy(ones16_hbm, ones_v)
        plsc.subcore_barrier()

        base = (c * NS + s) * EW

        def body(i, carry):
            off = pl.multiple_of(base + i * CH, 8)
            pltpu.sync_copy(dst_hbm.at[pl.ds(off, CH)], didx)
            pltpu.sync_copy(ones_v, acc16.at[didx], add=True)
            return carry

        lax.fori_loop(0, NCH, body, 0)
        plsc.subcore_barrier()
        pltpu.sync_copy(acc16.at[pl.ds(r0, ROWS_T)], out_hbm.at[c].at[pl.ds(r0, ROWS_T)])

    pl.run_scoped(inner, pltpu.VMEM_SHARED((N, 16), jnp.float32))


@functools.partial(
    pl.kernel,
    out_type=jax.ShapeDtypeStruct((NC, N, D), jnp.float32),
    mesh=_sc_mesh(),
    scratch_types=[
        pltpu.VMEM((CH,), jnp.int32),
        pltpu.VMEM((CH,), jnp.int32),
        pltpu.VMEM((CH, D), jnp.float32),
        pltpu.SemaphoreType.DMA,
    ],
)
def _sc_edge_agg(hp_hbm, src_hbm, dst_hbm, zeros_hbm, out_hbm, sidx, didx, rows, sem):
    c = lax.axis_index("c")
    s = lax.axis_index("s")

    def inner(acc):
        r0 = s * ROWS_T
        pltpu.sync_copy(zeros_hbm.at[pl.ds(r0, ROWS_T)], acc.at[pl.ds(r0, ROWS_T)])
        plsc.subcore_barrier()

        base = (c * NS + s) * EW

        def body(i, carry):
            off = pl.multiple_of(base + i * CH, 8)
            pltpu.sync_copy(src_hbm.at[pl.ds(off, CH)], sidx)
            pltpu.sync_copy(dst_hbm.at[pl.ds(off, CH)], didx)
            pltpu.async_copy(hp_hbm.at[sidx], rows, sem).wait()
            pltpu.sync_copy(rows, acc.at[didx], add=True)
            return carry

        lax.fori_loop(0, NCH, body, 0)
        plsc.subcore_barrier()
        pltpu.sync_copy(acc.at[pl.ds(r0, ROWS_T)], out_hbm.at[c].at[pl.ds(r0, ROWS_T)])

    pl.run_scoped(inner, pltpu.VMEM_SHARED((N, D), jnp.float32))


@functools.partial(
    pl.kernel,
    out_type=[
        jax.ShapeDtypeStruct((NW, G, D), jnp.float32),
        jax.ShapeDtypeStruct((NW, G, D), jnp.float32),
        jax.ShapeDtypeStruct((NW, G, 16), jnp.float32),
    ],
    mesh=_sc_mesh(),
    scratch_types=[
        pltpu.VMEM((RW, D), jnp.float32),
        pltpu.VMEM((RW,), jnp.int32),
        pltpu.VMEM((G, D), jnp.float32),
        pltpu.VMEM((G, D), jnp.float32),
        pltpu.VMEM((G, 16), jnp.float32),
    ],
)
def _sc_readout(h5_hbm, b_hbm, omax, osum, ocnt, hbuf, bidx, macc, sacc, cacc):
    c = lax.axis_index("c")
    s = lax.axis_index("s")
    w = c * NS + s
    start = w * RW
    nrows = jnp.where(w < NW - 1, RW, N - (NW - 1) * RW)
    nchunks = nrows // RCH

    neg_inf = jnp.full((16,), -jnp.inf, dtype=jnp.float32)
    zeros_v = jnp.zeros((16,), dtype=jnp.float32)
    ones_v = jnp.ones((16,), dtype=jnp.float32)

    def init_body(g, carry):
        for j in range(D // 16):
            macc[g, pl.ds(16 * j, 16)] = neg_inf
            sacc[g, pl.ds(16 * j, 16)] = zeros_v
        cacc[g, pl.ds(0, 16)] = zeros_v
        return carry

    lax.fori_loop(0, G, init_body, 0)

    def copy_body(j, carry):
        off = pl.multiple_of(start + j * RCH, 8)
        pltpu.sync_copy(h5_hbm.at[pl.ds(off, RCH)], hbuf.at[pl.ds(j * RCH, RCH)])
        pltpu.sync_copy(b_hbm.at[pl.ds(off, RCH)], bidx.at[pl.ds(j * RCH, RCH)])
        return carry

    lax.fori_loop(0, nchunks, copy_body, 0)

    def row_body(i, carry):
        g = bidx[i]
        for j in range(D // 16):
            v = hbuf[i, pl.ds(16 * j, 16)]
            macc[g, pl.ds(16 * j, 16)] = jnp.maximum(macc[g, pl.ds(16 * j, 16)], v)
            sacc[g, pl.ds(16 * j, 16)] = sacc[g, pl.ds(16 * j, 16)] + v
        cacc[g, pl.ds(0, 16)] = cacc[g, pl.ds(0, 16)] + ones_v
        return carry

    lax.fori_loop(0, nrows, row_body, 0)

    pltpu.sync_copy(macc, omax.at[w])
    pltpu.sync_copy(sacc, osum.at[w])
    pltpu.sync_copy(cacc, ocnt.at[w])


BLK = 1000


def _dinv_from(dega, degb):
    d = dega[:, :1] + degb[:, :1] + 1.0
    return lax.rsqrt(d)


def _tc0_body(x_ref, w_ref, dega_ref, degb_ref, o_ref):
    dinv = _dinv_from(dega_ref[...], degb_ref[...])
    o_ref[...] = dinv * jnp.dot(x_ref[...], w_ref[...],
                                preferred_element_type=jnp.float32)


def _tcmid_body(agga_ref, aggb_ref, hp_ref, dega_ref, degb_ref, b_ref, w_ref, o_ref):
    dinv = _dinv_from(dega_ref[...], degb_ref[...])
    t = dinv * (agga_ref[...] + aggb_ref[...] + hp_ref[...]) + b_ref[...]
    t = jnp.maximum(t, 0.0)
    o_ref[...] = dinv * jnp.dot(t, w_ref[...], preferred_element_type=jnp.float32)


def _tclast_body(agga_ref, aggb_ref, hp_ref, dega_ref, degb_ref, b_ref, o_ref):
    dinv = _dinv_from(dega_ref[...], degb_ref[...])
    t = dinv * (agga_ref[...] + aggb_ref[...] + hp_ref[...]) + b_ref[...]
    o_ref[...] = jnp.maximum(t, 0.0)


def _row_spec(width):
    return pl.BlockSpec((BLK, width), lambda i: (i, 0))


def _full_spec(shape):
    return pl.BlockSpec(shape, lambda i: (0, 0))


def _tc0(x, W0, dega, degb):
    return pl.pallas_call(
        _tc0_body,
        grid=(N // BLK,),
        in_specs=[_row_spec(D), _full_spec((D, D)), _row_spec(16), _row_spec(16)],
        out_specs=_row_spec(D),
        out_shape=jax.ShapeDtypeStruct((N, D), jnp.float32),
    )(x, W0, dega, degb)


def _tcmid(agga, aggb, hp, dega, degb, b, W):
    return pl.pallas_call(
        _tcmid_body,
        grid=(N // BLK,),
        in_specs=[_row_spec(D), _row_spec(D), _row_spec(D), _row_spec(16),
                  _row_spec(16), _full_spec((1, D)), _full_spec((D, D))],
        out_specs=_row_spec(D),
        out_shape=jax.ShapeDtypeStruct((N, D), jnp.float32),
    )(agga, aggb, hp, dega, degb, b, W)


def _tclast(agga, aggb, hp, dega, degb, b):
    return pl.pallas_call(
        _tclast_body,
        grid=(N // BLK,),
        in_specs=[_row_spec(D), _row_spec(D), _row_spec(D), _row_spec(16),
                  _row_spec(16), _full_spec((1, D))],
        out_specs=_row_spec(D),
        out_shape=jax.ShapeDtypeStruct((N, D), jnp.float32),
    )(agga, aggb, hp, dega, degb, b)


def _mlp_body(pmax_ref, psum_ref, pcnt_ref, w1_ref, b1_ref, w2_ref, b2_ref, o_ref):
    gmax = jnp.max(pmax_ref[...], axis=0)
    gsum = jnp.sum(psum_ref[...], axis=0)
    cnt = jnp.sum(pcnt_ref[...], axis=0)[:, :1]
    gmean = gsum / jnp.maximum(cnt, 1.0)
    hcat = jnp.concatenate([gmax, gmean], axis=1)
    h1 = hcat @ w1_ref[...] + b1_ref[...]
    h1 = jnp.maximum(h1, 0.0)
    o_ref[...] = h1 @ w2_ref[...] + b2_ref[...]


def _tc_mlp(pmax, psum, pcnt, out1_W, out1_b, out2_Wp, out2_bp):
    return pl.pallas_call(
        _mlp_body,
        out_shape=jax.ShapeDtypeStruct((G, D), jnp.float32),
    )(pmax, psum, pcnt, out1_W, out1_b, out2_Wp, out2_bp)


def kernel(x, edge_index, batch_index, W0, b0, W1, b1, W2, b2, W3, b3, W4, b4,
           out1_W, out1_b, out2_W, out2_b):
    assert x.shape == (N, D) and edge_index.shape == (2, E)

    src = edge_index[0].astype(jnp.int32)
    dst = edge_index[1].astype(jnp.int32)
    bidx = batch_index.astype(jnp.int32)

    zeros_nd = jnp.zeros((N, D), jnp.float32)
    zeros16 = jnp.zeros((N, 16), jnp.float32)
    ones16 = jnp.ones((CH, 16), jnp.float32)

    deg2 = _sc_degree(dst, zeros16, ones16)
    dega, degb = deg2[0], deg2[1]

    hp = _tc0(x, W0, dega, degb)
    bs = [b0, b1, b2, b3]
    Ws = [W1, W2, W3, W4]
    for layer in range(4):
        agg = _sc_edge_agg(hp, src, dst, zeros_nd)
        hp = _tcmid(agg[0], agg[1], hp, dega, degb, bs[layer].reshape(1, D), Ws[layer])
    agg = _sc_edge_agg(hp, src, dst, zeros_nd)
    h5 = _tclast(agg[0], agg[1], hp, dega, degb, b4.reshape(1, D))

    pmax, psum, pcnt = _sc_readout(h5, bidx)

    out2_Wp = jnp.pad(out2_W, ((0, 0), (0, D - out2_W.shape[1])))
    out2_bp = jnp.pad(out2_b, (0, D - out2_b.shape[0])).reshape(1, D)
    o = _tc_mlp(pmax, psum, pcnt, out1_W, out1_b.reshape(1, D), out2_Wp, out2_bp)
    return o[:, :1]


# speedup vs baseline: 12.9433x; 12.9433x over previous
import functools

import jax
import jax.numpy as jnp
from jax import lax
from jax.experimental import pallas as pl
from jax.experimental.pallas import tpu as pltpu
from jax.experimental.pallas import tpu_sc as plsc

N = 10000
E = 320000
D = 128
G = 64

NC = 2
NS = 16
NW = NC * NS

CH = 80
EW = E // NW
NCH = EW // CH
ROWS_A = 632
ROWS_B = N - (NS - 1) * ROWS_A

RW = 320
RCH = 80


def _sc_mesh():
    return plsc.VectorSubcoreMesh(core_axis_name="c", subcore_axis_name="s")


def _slab_copy(s, src_ref, dst_ref):
    @pl.when(s < NS - 1)
    def _():
        off = pl.multiple_of(s * ROWS_A, 8)
        pltpu.sync_copy(src_ref.at[pl.ds(off, ROWS_A)], dst_ref.at[pl.ds(off, ROWS_A)])

    @pl.when(s == NS - 1)
    def _():
        off = (NS - 1) * ROWS_A
        pltpu.sync_copy(src_ref.at[pl.ds(off, ROWS_B)], dst_ref.at[pl.ds(off, ROWS_B)])


@functools.partial(
    pl.kernel,
    out_type=jax.ShapeDtypeStruct((NC, N, D), jnp.float32),
    mesh=_sc_mesh(),
    scratch_types=[
        pltpu.VMEM((NCH, CH), jnp.int32),
        pltpu.VMEM((CH, D), jnp.float32),
        pltpu.VMEM_SHARED((N, D), jnp.float32),
    ],
)
def _sc_degree(dst3_hbm, zeros_hbm, ones_hbm, out_hbm, didx, ones_v, acc):
    c = lax.axis_index("c")
    s = lax.axis_index("s")
    w = c * NS + s

    _slab_copy(s, zeros_hbm, acc)
    pltpu.sync_copy(ones_hbm, ones_v)
    pltpu.sync_copy(dst3_hbm.at[w], didx)
    plsc.subcore_barrier()

    def body(i, carry):
        pltpu.sync_copy(ones_v, acc.at[didx.at[i]], add=True)
        return carry

    lax.fori_loop(0, NCH, body, 0)
    plsc.subcore_barrier()
    _slab_copy(s, acc, out_hbm.at[c])


@functools.partial(
    pl.kernel,
    out_type=jax.ShapeDtypeStruct((NC, N, D), jnp.float32),
    mesh=_sc_mesh(),
    scratch_types=[
        pltpu.VMEM((NCH, CH), jnp.int32),
        pltpu.VMEM((NCH, CH), jnp.int32),
        pltpu.VMEM((CH, D), jnp.float32),
        pltpu.SemaphoreType.DMA,
        pltpu.VMEM_SHARED((N, D), jnp.float32),
    ],
)
def _sc_edge_agg(hp_hbm, src3_hbm, dst3_hbm, zeros_hbm, out_hbm, sidx, didx, rows, sem, acc):
    c = lax.axis_index("c")
    s = lax.axis_index("s")
    w = c * NS + s

    _slab_copy(s, zeros_hbm, acc)
    pltpu.sync_copy(src3_hbm.at[w], sidx)
    pltpu.sync_copy(dst3_hbm.at[w], didx)
    plsc.subcore_barrier()

    def body(i, carry):
        pltpu.async_copy(hp_hbm.at[sidx.at[i]], rows, sem).wait()
        pltpu.sync_copy(rows, acc.at[didx.at[i]], add=True)
        return carry

    lax.fori_loop(0, NCH, body, 0)
    plsc.subcore_barrier()
    _slab_copy(s, acc, out_hbm.at[c])


@functools.partial(
    pl.kernel,
    out_type=[
        jax.ShapeDtypeStruct((NW, G, D), jnp.float32),
        jax.ShapeDtypeStruct((NW, G, D), jnp.float32),
        jax.ShapeDtypeStruct((NW, G, 16), jnp.float32),
    ],
    mesh=_sc_mesh(),
    scratch_types=[
        pltpu.VMEM((RW, D), jnp.float32),
        pltpu.VMEM((RW + 16,), jnp.int32),
        pltpu.VMEM((G, D), jnp.float32),
        pltpu.VMEM((G, D), jnp.float32),
        pltpu.VMEM((G, 16), jnp.float32),
    ],
)
def _sc_readout(h5_hbm, b_hbm, omax, osum, ocnt, hbuf, bidx, macc, sacc, cacc):
    c = lax.axis_index("c")
    s = lax.axis_index("s")
    w = c * NS + s
    start = w * RW
    nrows = jnp.where(w < NW - 1, RW, N - (NW - 1) * RW)
    nchunks = nrows // RCH

    neg_inf = jnp.full((16,), -jnp.inf, dtype=jnp.float32)
    zeros_v = jnp.zeros((16,), dtype=jnp.float32)
    ones_v = jnp.ones((16,), dtype=jnp.float32)

    def init_body(g, carry):
        for j in range(D // 16):
            macc[g, pl.ds(16 * j, 16)] = neg_inf
            sacc[g, pl.ds(16 * j, 16)] = zeros_v
        cacc[g, pl.ds(0, 16)] = zeros_v
        return carry

    lax.fori_loop(0, G, init_body, 0)

    def copy_body(j, carry):
        off = pl.multiple_of(start + j * RCH, 8)
        pltpu.sync_copy(h5_hbm.at[pl.ds(off, RCH)], hbuf.at[pl.ds(j * RCH, RCH)])
        pltpu.sync_copy(b_hbm.at[pl.ds(off, RCH)], bidx.at[pl.ds(j * RCH, RCH)])
        return carry

    lax.fori_loop(0, nchunks, copy_body, 0)

    def row_body(i, carry):
        g = bidx[pl.ds(i, 16)][0]
        for j in range(D // 16):
            v = hbuf[i, pl.ds(16 * j, 16)]
            macc[g, pl.ds(16 * j, 16)] = jnp.maximum(macc[g, pl.ds(16 * j, 16)], v)
            sacc[g, pl.ds(16 * j, 16)] = sacc[g, pl.ds(16 * j, 16)] + v
        cacc[g, pl.ds(0, 16)] = cacc[g, pl.ds(0, 16)] + ones_v
        return carry

    lax.fori_loop(0, nrows, row_body, 0)

    pltpu.sync_copy(macc, omax.at[w])
    pltpu.sync_copy(sacc, osum.at[w])
    pltpu.sync_copy(cacc, ocnt.at[w])


BLK = 1000


def _dinv_from(dega, degb):
    d = dega[:, :1] + degb[:, :1] + 1.0
    return lax.rsqrt(d)


def _tc0_body(x_ref, w_ref, dega_ref, degb_ref, o_ref):
    dinv = _dinv_from(dega_ref[...], degb_ref[...])
    o_ref[...] = dinv * jnp.dot(x_ref[...], w_ref[...],
                                preferred_element_type=jnp.float32)


def _tcmid_body(agga_ref, aggb_ref, hp_ref, dega_ref, degb_ref, b_ref, w_ref, o_ref):
    dinv = _dinv_from(dega_ref[...], degb_ref[...])
    t = dinv * (agga_ref[...] + aggb_ref[...] + hp_ref[...]) + b_ref[...]
    t = jnp.maximum(t, 0.0)
    o_ref[...] = dinv * jnp.dot(t, w_ref[...], preferred_element_type=jnp.float32)


def _tclast_body(agga_ref, aggb_ref, hp_ref, dega_ref, degb_ref, b_ref, o_ref):
    dinv = _dinv_from(dega_ref[...], degb_ref[...])
    t = dinv * (agga_ref[...] + aggb_ref[...] + hp_ref[...]) + b_ref[...]
    o_ref[...] = jnp.maximum(t, 0.0)


def _row_spec(width):
    return pl.BlockSpec((BLK, width), lambda i: (i, 0))


def _full_spec(shape):
    return pl.BlockSpec(shape, lambda i: (0, 0))


def _tc0(x, W0, dega, degb):
    return pl.pallas_call(
        _tc0_body,
        grid=(N // BLK,),
        in_specs=[_row_spec(D), _full_spec((D, D)), _row_spec(16), _row_spec(16)],
        out_specs=_row_spec(D),
        out_shape=jax.ShapeDtypeStruct((N, D), jnp.float32),
    )(x, W0, dega, degb)


def _tcmid(agga, aggb, hp, dega, degb, b, W):
    return pl.pallas_call(
        _tcmid_body,
        grid=(N // BLK,),
        in_specs=[_row_spec(D), _row_spec(D), _row_spec(D), _row_spec(16),
                  _row_spec(16), _full_spec((1, D)), _full_spec((D, D))],
        out_specs=_row_spec(D),
        out_shape=jax.ShapeDtypeStruct((N, D), jnp.float32),
    )(agga, aggb, hp, dega, degb, b, W)


def _tclast(agga, aggb, hp, dega, degb, b):
    return pl.pallas_call(
        _tclast_body,
        grid=(N // BLK,),
        in_specs=[_row_spec(D), _row_spec(D), _row_spec(D), _row_spec(16),
                  _row_spec(16), _full_spec((1, D))],
        out_specs=_row_spec(D),
        out_shape=jax.ShapeDtypeStruct((N, D), jnp.float32),
    )(agga, aggb, hp, dega, degb, b)


def _mlp_body(pmax_ref, psum_ref, pcnt_ref, w1_ref, b1_ref, w2_ref, b2_ref, o_ref):
    gmax = jnp.max(pmax_ref[...], axis=0)
    gsum = jnp.sum(psum_ref[...], axis=0)
    cnt = jnp.sum(pcnt_ref[...], axis=0)[:, :1]
    gmean = gsum / jnp.maximum(cnt, 1.0)
    hcat = jnp.concatenate([gmax, gmean], axis=1)
    h1 = hcat @ w1_ref[...] + b1_ref[...]
    h1 = jnp.maximum(h1, 0.0)
    o_ref[...] = h1 @ w2_ref[...] + b2_ref[...]


def _tc_mlp(pmax, psum, pcnt, out1_W, out1_b, out2_Wp, out2_bp):
    return pl.pallas_call(
        _mlp_body,
        out_shape=jax.ShapeDtypeStruct((G, D), jnp.float32),
    )(pmax, psum, pcnt, out1_W, out1_b, out2_Wp, out2_bp)


_USE_SC_DEG = True
_USE_SC_AGG = True
_USE_SC_READOUT = True


def kernel(x, edge_index, batch_index, W0, b0, W1, b1, W2, b2, W3, b3, W4, b4,
           out1_W, out1_b, out2_W, out2_b):
    assert x.shape == (N, D) and edge_index.shape == (2, E)

    src = edge_index[0].astype(jnp.int32)
    dst = edge_index[1].astype(jnp.int32)
    src3 = src.reshape(NW, NCH, CH)
    dst3 = dst.reshape(NW, NCH, CH)
    bidx = batch_index.astype(jnp.int32)

    zeros_nd = jnp.zeros((N, D), jnp.float32)
    ones_cd = jnp.ones((CH, D), jnp.float32)

    if _USE_SC_DEG:
        deg2 = _sc_degree(dst3, zeros_nd, ones_cd)
        dega, degb = deg2[0, :, :16], deg2[1, :, :16]
    else:
        deg = jax.ops.segment_sum(jnp.ones((E,), jnp.float32), dst, num_segments=N)
        dega = deg[:, None] * jnp.ones((1, 16), jnp.float32)
        degb = jnp.zeros((N, 16), jnp.float32)

    def do_agg(hp):
        if _USE_SC_AGG:
            agg = _sc_edge_agg(hp, src3, dst3, zeros_nd)
            return agg[0], agg[1]
        a = jax.ops.segment_sum(hp[src], dst, num_segments=N)
        return a, jnp.zeros_like(a)

    hp = _tc0(x, W0, dega, degb)
    bs = [b0, b1, b2, b3]
    Ws = [W1, W2, W3, W4]
    for layer in range(4):
        agga, aggb = do_agg(hp)
        hp = _tcmid(agga, aggb, hp, dega, degb, bs[layer].reshape(1, D), Ws[layer])
    agga, aggb = do_agg(hp)
    h5 = _tclast(agga, aggb, hp, dega, degb, b4.reshape(1, D))

    if _USE_SC_READOUT:
        pmax, psum, pcnt = _sc_readout(h5, bidx)
    else:
        gmax = jax.ops.segment_max(h5, bidx, num_segments=G)
        gsum = jax.ops.segment_sum(h5, bidx, num_segments=G)
        cnt = jax.ops.segment_sum(jnp.ones((N,), jnp.float32), bidx, num_segments=G)
        pad = jnp.full((NW - 1, G, D), -jnp.inf, jnp.float32)
        pmax = jnp.concatenate([gmax[None], pad], axis=0)
        psum = jnp.concatenate([gsum[None], jnp.zeros((NW - 1, G, D), jnp.float32)], axis=0)
        pcnt = jnp.concatenate([cnt[None, :, None] * jnp.ones((1, 1, 16), jnp.float32),
                                jnp.zeros((NW - 1, G, 16), jnp.float32)], axis=0)

    out2_Wp = jnp.pad(out2_W, ((0, 0), (0, D - out2_W.shape[1])))
    out2_bp = jnp.pad(out2_b, (0, D - out2_b.shape[0])).reshape(1, D)
    o = _tc_mlp(pmax, psum, pcnt, out1_W, out1_b.reshape(1, D), out2_Wp, out2_bp)
    return o[:, :1]


# speedup vs baseline: 19.6638x; 1.5192x over previous
import functools

import jax
import jax.numpy as jnp
from jax import lax
from jax.experimental import pallas as pl
from jax.experimental.pallas import tpu as pltpu
from jax.experimental.pallas import tpu_sc as plsc

N = 10000
E = 320000
D = 128
G = 64

NC = 2
NS = 16
NW = NC * NS

CH = 80
EW = E // NW
NCH = EW // CH
NBUF = 2
GA = NBUF - 1
NCYC = NCH // NBUF
DQ = 4
ROWS_A = 632
ROWS_B = N - (NS - 1) * ROWS_A

RW = 320
RCH = 80


def _sc_mesh():
    return plsc.VectorSubcoreMesh(core_axis_name="c", subcore_axis_name="s")


def _slab_copy(s, src_ref, dst_ref):
    @pl.when(s < NS - 1)
    def _():
        off = pl.multiple_of(s * ROWS_A, 8)
        pltpu.sync_copy(src_ref.at[pl.ds(off, ROWS_A)], dst_ref.at[pl.ds(off, ROWS_A)])

    @pl.when(s == NS - 1)
    def _():
        off = (NS - 1) * ROWS_A
        pltpu.sync_copy(src_ref.at[pl.ds(off, ROWS_B)], dst_ref.at[pl.ds(off, ROWS_B)])


@functools.partial(
    pl.kernel,
    out_type=jax.ShapeDtypeStruct((NC, N, D), jnp.float32),
    mesh=_sc_mesh(),
    scratch_types=[
        pltpu.VMEM_SHARED((N, D), jnp.float32),
        pltpu.SemaphoreType.DMA,
        pltpu.SemaphoreType.DMA,
        pltpu.SemaphoreType.DMA,
        pltpu.SemaphoreType.DMA,
    ],
)
def _sc_degree(dst3_hbm, zeros_hbm, ones_hbm, out_hbm, acc, s0, s1, s2, s3):
    c = lax.axis_index("c")
    s = lax.axis_index("s")
    w = c * NS + s
    ssem = [s0, s1, s2, s3]

    def inner(didx, ones_v):
        _slab_copy(s, zeros_hbm, acc)
        pltpu.sync_copy(ones_hbm, ones_v)
        pltpu.sync_copy(dst3_hbm.at[w], didx)
        plsc.subcore_barrier()

        def visit(v, b):
            @pl.when(v >= DQ)
            def _():
                pltpu.make_async_copy(
                    ones_v, acc.at[didx.at[v - DQ]], ssem[b]).wait()

            pltpu.async_copy(ones_v, acc.at[didx.at[v]], ssem[b], add=True)

        def cycle(k, carry):
            for b in range(DQ):
                visit(k * DQ + b, b)
            return carry

        ncyc = NCH // DQ
        lax.fori_loop(0, ncyc, cycle, 0)
        for r in range(NCH - ncyc * DQ):
            visit(jnp.int32(ncyc * DQ + r), r)
        for r in range(DQ):
            v = NCH - DQ + r
            pltpu.make_async_copy(ones_v, acc.at[didx.at[v]], ssem[v % DQ]).wait()
        plsc.subcore_barrier()
        _slab_copy(s, acc, out_hbm.at[c])

    pl.run_scoped(
        inner,
        pltpu.VMEM((NCH, CH), jnp.int32),
        pltpu.VMEM((CH, D), jnp.float32),
    )


@functools.partial(
    pl.kernel,
    out_type=jax.ShapeDtypeStruct((NC, N, D), jnp.float32),
    mesh=_sc_mesh(),
    scratch_types=[
        pltpu.VMEM_SHARED((N, D), jnp.float32),
        pltpu.SemaphoreType.DMA,
        pltpu.SemaphoreType.DMA,
        pltpu.SemaphoreType.DMA,
        pltpu.SemaphoreType.DMA,
    ],
)
def _sc_edge_agg(hp_hbm, src_hbm, dst3_hbm, zeros_hbm, out_hbm,
                 acc, g0, g1, ss0, ss1):
    c = lax.axis_index("c")
    s = lax.axis_index("s")
    w = c * NS + s
    gsem = [g0, g1]
    ssem = [ss0, ss1]

    def inner(sidx, didx, r0, r1):
        rows = [r0, r1]

        _slab_copy(s, zeros_hbm, acc)
        pltpu.sync_copy(src_hbm.at[pl.ds(pl.multiple_of(w * EW, 8), EW)], sidx)
        pltpu.sync_copy(dst3_hbm.at[w], didx)
        plsc.subcore_barrier()

        def chunk_idx(chunk):
            return sidx.at[pl.ds(pl.multiple_of(chunk * CH, 8), CH)]

        def gather_start(chunk, b):
            pltpu.async_copy(hp_hbm.at[chunk_idx(chunk)], rows[b], gsem[b])

        gather_start(0, 0)

        def visit(v, b):
            bg = (b + GA) % NBUF
            gc = v + GA

            @pl.when((v >= 1) & (gc < NCH))
            def _():
                pltpu.make_async_copy(
                    rows[bg], acc.at[didx.at[gc - NBUF]], ssem[bg]).wait()

            @pl.when(gc < NCH)
            def _():
                gather_start(gc, bg)

            pltpu.make_async_copy(hp_hbm.at[chunk_idx(v)], rows[b], gsem[b]).wait()
            pltpu.async_copy(rows[b], acc.at[didx.at[v]], ssem[b], add=True)

        def cycle(k, carry):
            for b in range(NBUF):
                visit(k * NBUF + b, b)
            return carry

        lax.fori_loop(0, NCYC, cycle, 0)
        for r in range(NCH - NCYC * NBUF):
            visit(jnp.int32(NCYC * NBUF + r), r)
        for r in range(NBUF):
            v = NCH - NBUF + r
            pltpu.make_async_copy(rows[v % NBUF], acc.at[didx.at[v]], ssem[v % NBUF]).wait()
        plsc.subcore_barrier()
        _slab_copy(s, acc, out_hbm.at[c])

    pl.run_scoped(
        inner,
        pltpu.VMEM((EW,), jnp.int32),
        pltpu.VMEM((NCH, CH), jnp.int32),
        pltpu.VMEM((CH, D), jnp.float32),
        pltpu.VMEM((CH, D), jnp.float32),
    )


@functools.partial(
    pl.kernel,
    out_type=[
        jax.ShapeDtypeStruct((NW, G, D), jnp.float32),
        jax.ShapeDtypeStruct((NW, G, D), jnp.float32),
        jax.ShapeDtypeStruct((NW, G, 16), jnp.float32),
    ],
    mesh=_sc_mesh(),
    scratch_types=[
        pltpu.VMEM((RW, D), jnp.float32),
        pltpu.VMEM((RW + 16,), jnp.int32),
        pltpu.VMEM((G, D), jnp.float32),
        pltpu.VMEM((G, D), jnp.float32),
        pltpu.VMEM((G, 16), jnp.float32),
    ],
)
def _sc_readout(h5_hbm, b_hbm, omax, osum, ocnt, hbuf, bidx, macc, sacc, cacc):
    c = lax.axis_index("c")
    s = lax.axis_index("s")
    w = c * NS + s
    start = w * RW
    nrows = jnp.where(w < NW - 1, RW, N - (NW - 1) * RW)
    nchunks = nrows // RCH

    neg_inf = jnp.full((16,), -jnp.inf, dtype=jnp.float32)
    zeros_v = jnp.zeros((16,), dtype=jnp.float32)
    ones_v = jnp.ones((16,), dtype=jnp.float32)

    def init_body(g, carry):
        for j in range(D // 16):
            macc[g, pl.ds(16 * j, 16)] = neg_inf
            sacc[g, pl.ds(16 * j, 16)] = zeros_v
        cacc[g, pl.ds(0, 16)] = zeros_v
        return carry

    lax.fori_loop(0, G, init_body, 0)

    def copy_body(j, carry):
        off = pl.multiple_of(start + j * RCH, 8)
        pltpu.sync_copy(h5_hbm.at[pl.ds(off, RCH)], hbuf.at[pl.ds(j * RCH, RCH)])
        pltpu.sync_copy(b_hbm.at[pl.ds(off, RCH)], bidx.at[pl.ds(j * RCH, RCH)])
        return carry

    lax.fori_loop(0, nchunks, copy_body, 0)

    def row_body(i, carry):
        g = bidx[pl.ds(i, 16)][0]
        for j in range(D // 16):
            v = hbuf[i, pl.ds(16 * j, 16)]
            macc[g, pl.ds(16 * j, 16)] = jnp.maximum(macc[g, pl.ds(16 * j, 16)], v)
            sacc[g, pl.ds(16 * j, 16)] = sacc[g, pl.ds(16 * j, 16)] + v
        cacc[g, pl.ds(0, 16)] = cacc[g, pl.ds(0, 16)] + ones_v
        return carry

    lax.fori_loop(0, nrows, row_body, 0)

    pltpu.sync_copy(macc, omax.at[w])
    pltpu.sync_copy(sacc, osum.at[w])
    pltpu.sync_copy(cacc, ocnt.at[w])


BLK = 1000


def _dinv_from(dega, degb):
    d = dega[:, :1] + degb[:, :1] + 1.0
    return lax.rsqrt(d)


def _tc0_body(x_ref, w_ref, dega_ref, degb_ref, o_ref):
    dinv = _dinv_from(dega_ref[...], degb_ref[...])
    o_ref[...] = dinv * jnp.dot(x_ref[...], w_ref[...],
                                preferred_element_type=jnp.float32)


def _tcmid_body(agga_ref, aggb_ref, hp_ref, dega_ref, degb_ref, b_ref, w_ref, o_ref):
    dinv = _dinv_from(dega_ref[...], degb_ref[...])
    t = dinv * (agga_ref[...] + aggb_ref[...] + hp_ref[...]) + b_ref[...]
    t = jnp.maximum(t, 0.0)
    o_ref[...] = dinv * jnp.dot(t, w_ref[...], preferred_element_type=jnp.float32)


def _tclast_body(agga_ref, aggb_ref, hp_ref, dega_ref, degb_ref, b_ref, o_ref):
    dinv = _dinv_from(dega_ref[...], degb_ref[...])
    t = dinv * (agga_ref[...] + aggb_ref[...] + hp_ref[...]) + b_ref[...]
    o_ref[...] = jnp.maximum(t, 0.0)


def _row_spec(width):
    return pl.BlockSpec((BLK, width), lambda i: (i, 0))


def _full_spec(shape):
    return pl.BlockSpec(shape, lambda i: (0, 0))


def _tc0(x, W0, dega, degb):
    return pl.pallas_call(
        _tc0_body,
        grid=(N // BLK,),
        in_specs=[_row_spec(D), _full_spec((D, D)), _row_spec(16), _row_spec(16)],
        out_specs=_row_spec(D),
        out_shape=jax.ShapeDtypeStruct((N, D), jnp.float32),
    )(x, W0, dega, degb)


def _tcmid(agga, aggb, hp, dega, degb, b, W):
    return pl.pallas_call(
        _tcmid_body,
        grid=(N // BLK,),
        in_specs=[_row_spec(D), _row_spec(D), _row_spec(D), _row_spec(16),
                  _row_spec(16), _full_spec((1, D)), _full_spec((D, D))],
        out_specs=_row_spec(D),
        out_shape=jax.ShapeDtypeStruct((N, D), jnp.float32),
    )(agga, aggb, hp, dega, degb, b, W)


def _tclast(agga, aggb, hp, dega, degb, b):
    return pl.pallas_call(
        _tclast_body,
        grid=(N // BLK,),
        in_specs=[_row_spec(D), _row_spec(D), _row_spec(D), _row_spec(16),
                  _row_spec(16), _full_spec((1, D))],
        out_specs=_row_spec(D),
        out_shape=jax.ShapeDtypeStruct((N, D), jnp.float32),
    )(agga, aggb, hp, dega, degb, b)


def _mlp_body(pmax_ref, psum_ref, pcnt_ref, w1_ref, b1_ref, w2_ref, b2_ref, o_ref):
    gmax = jnp.max(pmax_ref[...], axis=0)
    gsum = jnp.sum(psum_ref[...], axis=0)
    cnt = jnp.sum(pcnt_ref[...], axis=0)[:, :1]
    gmean = gsum / jnp.maximum(cnt, 1.0)
    hcat = jnp.concatenate([gmax, gmean], axis=1)
    h1 = hcat @ w1_ref[...] + b1_ref[...]
    h1 = jnp.maximum(h1, 0.0)
    o_ref[...] = h1 @ w2_ref[...] + b2_ref[...]


def _tc_mlp(pmax, psum, pcnt, out1_W, out1_b, out2_Wp, out2_bp):
    return pl.pallas_call(
        _mlp_body,
        out_shape=jax.ShapeDtypeStruct((G, D), jnp.float32),
    )(pmax, psum, pcnt, out1_W, out1_b, out2_Wp, out2_bp)


_USE_SC_DEG = True
_USE_SC_AGG = True
_USE_SC_READOUT = True


def kernel(x, edge_index, batch_index, W0, b0, W1, b1, W2, b2, W3, b3, W4, b4,
           out1_W, out1_b, out2_W, out2_b):
    assert x.shape == (N, D) and edge_index.shape == (2, E)

    src = edge_index[0].astype(jnp.int32)
    dst = edge_index[1].astype(jnp.int32)
    src3 = src.reshape(NW, NCH, CH)
    dst3 = dst.reshape(NW, NCH, CH)
    bidx = batch_index.astype(jnp.int32)

    zeros_nd = jnp.zeros((N, D), jnp.float32)
    ones_cd = jnp.ones((CH, D), jnp.float32)

    if _USE_SC_DEG:
        deg2 = _sc_degree(dst3, zeros_nd, ones_cd)
        dega, degb = deg2[0, :, :16], deg2[1, :, :16]
    else:
        deg = jax.ops.segment_sum(jnp.ones((E,), jnp.float32), dst, num_segments=N)
        dega = deg[:, None] * jnp.ones((1, 16), jnp.float32)
        degb = jnp.zeros((N, 16), jnp.float32)

    def do_agg(hp):
        if _USE_SC_AGG:
            agg = _sc_edge_agg(hp, src, dst3, zeros_nd)
            return agg[0], agg[1]
        a = jax.ops.segment_sum(hp[src], dst, num_segments=N)
        return a, jnp.zeros_like(a)

    hp = _tc0(x, W0, dega, degb)
    bs = [b0, b1, b2, b3]
    Ws = [W1, W2, W3, W4]
    for layer in range(4):
        agga, aggb = do_agg(hp)
        hp = _tcmid(agga, aggb, hp, dega, degb, bs[layer].reshape(1, D), Ws[layer])
    agga, aggb = do_agg(hp)
    h5 = _tclast(agga, aggb, hp, dega, degb, b4.reshape(1, D))

    if _USE_SC_READOUT:
        pmax, psum, pcnt = _sc_readout(h5, bidx)
    else:
        gmax = jax.ops.segment_max(h5, bidx, num_segments=G)
        gsum = jax.ops.segment_sum(h5, bidx, num_segments=G)
        cnt = jax.ops.segment_sum(jnp.ones((N,), jnp.float32), bidx, num_segments=G)
        pad = jnp.full((NW - 1, G, D), -jnp.inf, jnp.float32)
        pmax = jnp.concatenate([gmax[None], pad], axis=0)
        psum = jnp.concatenate([gsum[None], jnp.zeros((NW - 1, G, D), jnp.float32)], axis=0)
        pcnt = jnp.concatenate([cnt[None, :, None] * jnp.ones((1, 1, 16), jnp.float32),
                                jnp.zeros((NW - 1, G, 16), jnp.float32)], axis=0)

    out2_Wp = jnp.pad(out2_W, ((0, 0), (0, D - out2_W.shape[1])))
    out2_bp = jnp.pad(out2_b, (0, D - out2_b.shape[0])).reshape(1, D)
    o = _tc_mlp(pmax, psum, pcnt, out1_W, out1_b.reshape(1, D), out2_Wp, out2_bp)
    return o[:, :1]
